# Initial kernel scaffold; baseline (speedup 1.0000x reference)
import functools

import jax
import jax.numpy as jnp
from jax import lax
from jax.experimental import pallas as pl
from jax.experimental.pallas import tpu as pltpu
from jax.experimental.pallas import tpu_sc as plsc

N = 50000
D = 256
S = 256
L = 16
NC = 2
NS = 16
NW = NC * NS
BR = 112
NBLK = (N + BR - 1) // BR
KMAX = (NBLK + NW - 1) // NW
FV = D // L

_mesh = plsc.VectorSubcoreMesh(
    core_axis_name="c", subcore_axis_name="s", num_cores=NC, num_subcores=NS)

_f32 = jnp.float32
_i32 = jnp.int32


def _stats_body(x_hbm, batch_hbm, psum, psq, pcnt,
                xbuf, sqbuf, idxbuf, onesbuf, zbuf, zcnt,
                ssum, ssq, scnt):
    cid = lax.axis_index("c")
    sid = lax.axis_index("s")
    wid = sid * NC + cid

    for r in range(16):
        for f in range(FV):
            zbuf[r, pl.ds(f * L, L)] = jnp.zeros((L,), _f32)
        zcnt[r, pl.ds(0, L)] = jnp.zeros((L,), _f32)
    base = sid * 16
    pltpu.sync_copy(zbuf, ssum.at[pl.ds(base, 16)])
    pltpu.sync_copy(zbuf, ssq.at[pl.ds(base, 16)])
    pltpu.sync_copy(zcnt, scnt.at[pl.ds(base, 16)])

    def _ones(r, _):
        onesbuf[r, pl.ds(0, L)] = jnp.ones((L,), _f32)
        return 0
    lax.fori_loop(0, BR, _ones, 0)
    plsc.subcore_barrier()

    def block_body(k, _):
        b = wid + NW * k

        @pl.when(b < NBLK)
        def _():
            start = pl.multiple_of(jnp.minimum(b * BR, N - BR), 8)
            pltpu.sync_copy(x_hbm.at[pl.ds(start, BR)], xbuf)
            pltpu.sync_copy(batch_hbm.at[pl.ds(start, BR)], idxbuf)
            vfrom = b * BR
            vto = jnp.minimum(vfrom + BR, N)
            for j in range(BR // L):
                pos = start + j * L + lax.iota(_i32, 16)
                v = idxbuf[pl.ds(j * L, L)]
                ok = (pos >= vfrom) & (pos < vto)
                idxbuf[pl.ds(j * L, L)] = jnp.where(ok, v, S)

            def row_body(r, _):
                for f in range(FV):
                    xv = xbuf[r, pl.ds(f * L, L)]
                    sqbuf[r, pl.ds(f * L, L)] = xv * xv
                return 0
            lax.fori_loop(0, BR, row_body, 0)

            pltpu.sync_copy(xbuf, ssum.at[idxbuf], add=True)
            pltpu.sync_copy(sqbuf, ssq.at[idxbuf], add=True)
            pltpu.sync_copy(onesbuf, scnt.at[idxbuf], add=True)
        return 0

    lax.fori_loop(0, KMAX, block_body, 0)
    plsc.subcore_barrier()

    pltpu.sync_copy(ssum.at[pl.ds(base, 16)], psum.at[cid, pl.ds(base, 16)])
    pltpu.sync_copy(ssq.at[pl.ds(base, 16)], psq.at[cid, pl.ds(base, 16)])
    pltpu.sync_copy(scnt.at[pl.ds(base, 16)], pcnt.at[cid, pl.ds(base, 16)])


_stats = functools.partial(
    pl.kernel,
    out_type=(
        jax.ShapeDtypeStruct((NC, S, D), _f32),
        jax.ShapeDtypeStruct((NC, S, D), _f32),
        jax.ShapeDtypeStruct((NC, S, L), _f32),
    ),
    mesh=_mesh,
    scratch_types=[
        pltpu.VMEM((BR, D), _f32),
        pltpu.VMEM((BR, D), _f32),
        pltpu.VMEM((BR,), _i32),
        pltpu.VMEM((BR, L), _f32),
        pltpu.VMEM((16, D), _f32),
        pltpu.VMEM((16, L), _f32),
        pltpu.VMEM_SHARED((S + 1, D), _f32),
        pltpu.VMEM_SHARED((S + 1, D), _f32),
        pltpu.VMEM_SHARED((S + 1, L), _f32),
    ],
)(_stats_body)


def _finalize_body(psum_ref, psq_ref, pcnt_ref, scale_ref, bias_ref, ab_ref):
    s = psum_ref[0] + psum_ref[1]
    q = psq_ref[0] + psq_ref[1]
    c = pcnt_ref[0, :, 0:1] + pcnt_ref[1, :, 0:1]
    c_safe = jnp.maximum(c, 1.0)
    mean = s / c_safe
    denom = jnp.maximum(c - 1.0, 1.0)
    var = jnp.maximum((q - c * mean * mean) / denom, 0.0)
    std = jnp.sqrt(var)
    a = scale_ref[...][None, :] / (std + 1e-5)
    bb = bias_ref[...][None, :] - mean * a
    ab_ref[:, 0:D] = a
    ab_ref[:, D:2 * D] = bb


def _finalize(psum, psq, pcnt, scale, bias):
    return pl.pallas_call(
        _finalize_body,
        out_shape=jax.ShapeDtypeStruct((S, 2 * D), _f32),
    )(psum, psq, pcnt, scale, bias)


def _norm_body(x_hbm, batch_hbm, ab_hbm, out_hbm, xbuf, idxbuf, abbuf, sem):
    cid = lax.axis_index("c")
    sid = lax.axis_index("s")
    wid = sid * NC + cid

    def block_body(k, _):
        b = wid + NW * k

        @pl.when(b < NBLK)
        def _():
            start = pl.multiple_of(jnp.minimum(b * BR, N - BR), 8)
            pltpu.sync_copy(x_hbm.at[pl.ds(start, BR)], xbuf)
            pltpu.sync_copy(batch_hbm.at[pl.ds(start, BR)], idxbuf)
            pltpu.async_copy(ab_hbm.at[idxbuf], abbuf, sem).wait()

            def row_body(r, _):
                for f in range(FV):
                    xv = xbuf[r, pl.ds(f * L, L)]
                    av = abbuf[r, pl.ds(f * L, L)]
                    bv = abbuf[r, pl.ds(D + f * L, L)]
                    xbuf[r, pl.ds(f * L, L)] = xv * av + bv
                return 0
            lax.fori_loop(0, BR, row_body, 0)

            pltpu.sync_copy(xbuf, out_hbm.at[pl.ds(start, BR)])
        return 0

    lax.fori_loop(0, KMAX, block_body, 0)


_norm = functools.partial(
    pl.kernel,
    out_type=jax.ShapeDtypeStruct((N, D), _f32),
    mesh=_mesh,
    scratch_types=[
        pltpu.VMEM((BR, D), _f32),
        pltpu.VMEM((BR,), _i32),
        pltpu.VMEM((BR, 2 * D), _f32),
        pltpu.SemaphoreType.DMA,
    ],
)(_norm_body)


def kernel(x, batch, scale, bias):
    psum, psq, pcnt = _stats(x, batch)
    ab = _finalize(psum, psq, pcnt, scale, bias)
    return _norm(x, batch, ab)


# speedup vs baseline: 2.0310x; 2.0310x over previous
import functools

import jax
import jax.numpy as jnp
from jax import lax
from jax.experimental import pallas as pl
from jax.experimental.pallas import tpu as pltpu
from jax.experimental.pallas import tpu_sc as plsc

N = 50000
D = 256
S = 256
L = 16
NC = 2
NS = 16
NW = NC * NS
BR = 112
NBLK = (N + BR - 1) // BR
KMAX = (NBLK + NW - 1) // NW
H = D // 2
FH = H // L
FV = D // L

_f32 = jnp.float32
_i32 = jnp.int32


def _stats_body(x_hbm, batch_hbm, psum, psq, pcnt,
                xbuf, idxbuf, asum, asq, acnt):
    cid = lax.axis_index("c")
    sid = lax.axis_index("s")
    wid = sid * NC + cid
    ones = jnp.ones((L,), _f32)

    for h in range(2):
        def zero_body(r, _):
            for f in range(FH):
                asum[r, pl.ds(f * L, L)] = jnp.zeros((L,), _f32)
                asq[r, pl.ds(f * L, L)] = jnp.zeros((L,), _f32)
            if h == 0:
                acnt[r, pl.ds(0, L)] = jnp.zeros((L,), _f32)
            return 0
        lax.fori_loop(0, S + 1, zero_body, 0)

        def block_body(k, _):
            b = wid + NW * k

            @pl.when(b < NBLK)
            def _():
                start = pl.multiple_of(jnp.minimum(b * BR, N - BR), 8)
                pltpu.sync_copy(
                    x_hbm.at[pl.ds(start, BR), pl.ds(h * H, H)], xbuf)
                pltpu.sync_copy(batch_hbm.at[pl.ds(start, BR)], idxbuf)
                vfrom = b * BR
                vto = jnp.minimum(vfrom + BR, N)

                def group_body(g, _):
                    bv = idxbuf[pl.ds(g * L, L)]
                    pos0 = start + g * L
                    for j in range(L):
                        pos = pos0 + j
                        valid = (pos >= vfrom) & (pos < vto)
                        seg = jnp.where(valid, bv[j], S)
                        r = g * L + j
                        for f in range(FH):
                            xv = xbuf[r, pl.ds(f * L, L)]
                            plsc.addupdate(asum.at[seg, pl.ds(f * L, L)], xv)
                            plsc.addupdate(asq.at[seg, pl.ds(f * L, L)],
                                           xv * xv)
                        if h == 0:
                            plsc.addupdate(acnt.at[seg, pl.ds(0, L)], ones)
                    return 0
                lax.fori_loop(0, BR // L, group_body, 0)
            return 0

        lax.fori_loop(0, KMAX, block_body, 0)

        pltpu.sync_copy(asum, psum.at[h, wid])
        pltpu.sync_copy(asq, psq.at[h, wid])
        if h == 0:
            pltpu.sync_copy(acnt, pcnt.at[wid])


@functools.cache
def _make_stats():
  mesh = plsc.VectorSubcoreMesh(
      core_axis_name="c", subcore_axis_name="s",
      num_cores=NC, num_subcores=NS)
  return functools.partial(
    pl.kernel,
    mesh=mesh,
    out_type=(
        jax.ShapeDtypeStruct((2, NW, S + 1, H), _f32),
        jax.ShapeDtypeStruct((2, NW, S + 1, H), _f32),
        jax.ShapeDtypeStruct((NW, S + 1, L), _f32),
    ),
    scratch_types=[
        pltpu.VMEM((BR, H), _f32),
        pltpu.VMEM((BR,), _i32),
        pltpu.VMEM((S + 1, H), _f32),
        pltpu.VMEM((S + 1, H), _f32),
        pltpu.VMEM((S + 1, L), _f32),
    ],
  )(_stats_body)


def _finalize_body(psum_ref, psq_ref, pcnt_ref, scale_ref, bias_ref, ab_ref):
    s0 = jnp.sum(psum_ref[0], axis=0)[:S]
    s1 = jnp.sum(psum_ref[1], axis=0)[:S]
    q0 = jnp.sum(psq_ref[0], axis=0)[:S]
    q1 = jnp.sum(psq_ref[1], axis=0)[:S]
    s = jnp.concatenate([s0, s1], axis=1)
    q = jnp.concatenate([q0, q1], axis=1)
    c = jnp.sum(pcnt_ref[...], axis=0)[:S, 0:1]
    c_safe = jnp.maximum(c, 1.0)
    mean = s / c_safe
    denom = jnp.maximum(c - 1.0, 1.0)
    var = jnp.maximum((q - c * mean * mean) / denom, 0.0)
    std = jnp.sqrt(var)
    a = scale_ref[...][None, :] / (std + 1e-5)
    bb = bias_ref[...][None, :] - mean * a
    ab_ref[:, 0:D] = a
    ab_ref[:, D:2 * D] = bb


def _finalize(psum, psq, pcnt, scale, bias):
    return pl.pallas_call(
        _finalize_body,
        out_shape=jax.ShapeDtypeStruct((S, 2 * D), _f32),
    )(psum, psq, pcnt, scale, bias)


def _norm_body(x_hbm, batch_hbm, ab_hbm, out_hbm, xbuf, idxbuf, abbuf, sem):
    cid = lax.axis_index("c")
    sid = lax.axis_index("s")
    wid = sid * NC + cid

    def block_body(k, _):
        b = wid + NW * k

        @pl.when(b < NBLK)
        def _():
            start = pl.multiple_of(jnp.minimum(b * BR, N - BR), 8)
            pltpu.sync_copy(x_hbm.at[pl.ds(start, BR)], xbuf)
            pltpu.sync_copy(batch_hbm.at[pl.ds(start, BR)], idxbuf)
            pltpu.async_copy(ab_hbm.at[idxbuf], abbuf, sem).wait()

            def row_body(r, _):
                for f in range(FV):
                    xv = xbuf[r, pl.ds(f * L, L)]
                    av = abbuf[r, pl.ds(f * L, L)]
                    bv = abbuf[r, pl.ds(D + f * L, L)]
                    xbuf[r, pl.ds(f * L, L)] = xv * av + bv
                return 0
            lax.fori_loop(0, BR, row_body, 0)

            pltpu.sync_copy(xbuf, out_hbm.at[pl.ds(start, BR)])
        return 0

    lax.fori_loop(0, KMAX, block_body, 0)


@functools.cache
def _make_norm():
  mesh = plsc.VectorSubcoreMesh(
      core_axis_name="c", subcore_axis_name="s",
      num_cores=NC, num_subcores=NS)
  return functools.partial(
    pl.kernel,
    mesh=mesh,
    out_type=jax.ShapeDtypeStruct((N, D), _f32),
    scratch_types=[
        pltpu.VMEM((BR, D), _f32),
        pltpu.VMEM((BR,), _i32),
        pltpu.VMEM((BR, 2 * D), _f32),
        pltpu.SemaphoreType.DMA,
    ],
  )(_norm_body)


def kernel(x, batch, scale, bias):
    psum, psq, pcnt = _make_stats()(x, batch)
    ab = _finalize(psum, psq, pcnt, scale, bias)
    return _make_norm()(x, batch, ab)


# speedup vs baseline: 2.6729x; 1.3160x over previous
import functools

import jax
import jax.numpy as jnp
from jax import lax
from jax.experimental import pallas as pl
from jax.experimental.pallas import tpu as pltpu
from jax.experimental.pallas import tpu_sc as plsc

N = 50000
D = 256
S = 256
L = 16
NC = 2
NS = 16
NW = NC * NS
BR = 112
NBLK = (N + BR - 1) // BR
KMAX = (NBLK + NW - 1) // NW
H = D // 2
FH = H // L
FV = D // L

_f32 = jnp.float32
_i32 = jnp.int32


def _stats_body(x_hbm, batch_hbm, psum, psq, pcnt,
                xbuf, idxbuf, asum, asq, acnt):
    cid = lax.axis_index("c")
    sid = lax.axis_index("s")
    wid = sid * NC + cid
    ones = jnp.ones((L,), _f32)

    for h in range(2):
        def zero_body(r, _):
            for f in range(FH):
                asum[r, pl.ds(f * L, L)] = jnp.zeros((L,), _f32)
                asq[r, pl.ds(f * L, L)] = jnp.zeros((L,), _f32)
            if h == 0:
                acnt[r, pl.ds(0, L)] = jnp.zeros((L,), _f32)
            return 0
        lax.fori_loop(0, S + 1, zero_body, 0)

        def block_body(k, _):
            b = wid + NW * k

            @pl.when(b < NBLK)
            def _():
                start = pl.multiple_of(jnp.minimum(b * BR, N - BR), 8)
                pltpu.sync_copy(
                    x_hbm.at[pl.ds(start, BR), pl.ds(h * H, H)], xbuf)
                pltpu.sync_copy(batch_hbm.at[pl.ds(start, BR)], idxbuf)
                vfrom = b * BR
                vto = jnp.minimum(vfrom + BR, N)

                def group_body(g, _):
                    bv = idxbuf[pl.ds(g * L, L)]
                    pos0 = start + g * L
                    for j in range(L):
                        pos = pos0 + j
                        valid = (pos >= vfrom) & (pos < vto)
                        seg = jnp.where(valid, bv[j], S)
                        r = g * L + j
                        for f in range(FH):
                            xv = xbuf[r, pl.ds(f * L, L)]
                            plsc.addupdate(asum.at[seg, pl.ds(f * L, L)], xv)
                            plsc.addupdate(asq.at[seg, pl.ds(f * L, L)],
                                           xv * xv)
                        if h == 0:
                            plsc.addupdate(acnt.at[seg, pl.ds(0, L)], ones)
                    return 0
                lax.fori_loop(0, BR // L, group_body, 0)
            return 0

        lax.fori_loop(0, KMAX, block_body, 0)

        pltpu.sync_copy(asum, psum.at[h, wid])
        pltpu.sync_copy(asq, psq.at[h, wid])
        if h == 0:
            pltpu.sync_copy(acnt, pcnt.at[wid])


@functools.cache
def _make_stats():
  mesh = plsc.VectorSubcoreMesh(
      core_axis_name="c", subcore_axis_name="s",
      num_cores=NC, num_subcores=NS)
  return functools.partial(
    pl.kernel,
    mesh=mesh,
    out_type=(
        jax.ShapeDtypeStruct((2, NW, S + 1, H), _f32),
        jax.ShapeDtypeStruct((2, NW, S + 1, H), _f32),
        jax.ShapeDtypeStruct((NW, S + 1, L), _f32),
    ),
    scratch_types=[
        pltpu.VMEM((BR, H), _f32),
        pltpu.VMEM((BR,), _i32),
        pltpu.VMEM((S + 1, H), _f32),
        pltpu.VMEM((S + 1, H), _f32),
        pltpu.VMEM((S + 1, L), _f32),
    ],
  )(_stats_body)


def _finalize_body(psum_ref, psq_ref, pcnt_ref, scale_ref, bias_ref, ab_ref):
    s0 = jnp.sum(psum_ref[0], axis=0)[:S]
    s1 = jnp.sum(psum_ref[1], axis=0)[:S]
    q0 = jnp.sum(psq_ref[0], axis=0)[:S]
    q1 = jnp.sum(psq_ref[1], axis=0)[:S]
    s = jnp.concatenate([s0, s1], axis=1)
    q = jnp.concatenate([q0, q1], axis=1)
    c = jnp.sum(pcnt_ref[...], axis=0)[:S, 0:1]
    c_safe = jnp.maximum(c, 1.0)
    mean = s / c_safe
    denom = jnp.maximum(c - 1.0, 1.0)
    var = jnp.maximum((q - c * mean * mean) / denom, 0.0)
    std = jnp.sqrt(var)
    a = scale_ref[...][None, :] / (std + 1e-5)
    bb = bias_ref[...][None, :] - mean * a
    ab_ref[:, 0:D] = a
    ab_ref[:, D:2 * D] = bb


def _finalize(psum, psq, pcnt, scale, bias):
    return pl.pallas_call(
        _finalize_body,
        out_shape=jax.ShapeDtypeStruct((S, 2 * D), _f32),
    )(psum, psq, pcnt, scale, bias)


RPT = 1568
NB = 56
NKB = RPT // NB
_norm_bufs = 2


def _norm_body(x_hbm, batch_hbm, ab_hbm, out_hbm,
               idxall, xb0, xb1, ab0, ab1, ob0, ob1,
               xs0, xs1, as0, as1, os0, os1):
    cid = lax.axis_index("c")
    sid = lax.axis_index("s")
    wid = sid * NC + cid
    xb = (xb0, xb1)
    abb = (ab0, ab1)
    ob = (ob0, ob1)
    xsem = (xs0, xs1)
    asem = (as0, as1)
    osem = (os0, os1)

    base = RPT * wid
    astart = pl.multiple_of(jnp.minimum(base, N - RPT), 8)
    pltpu.sync_copy(batch_hbm.at[pl.ds(astart, RPT)], idxall)
    nblk = (jnp.minimum(N - base, RPT) + NB - 1) // NB

    def pstart(k):
        return pl.multiple_of(jnp.minimum(base + NB * k, N - NB), 8)

    def issue_loads(k, i):
        ps = pstart(k)
        pltpu.async_copy(x_hbm.at[pl.ds(ps, NB)], xb[i], xsem[i])
        pltpu.async_copy(
            ab_hbm.at[idxall.at[pl.ds(ps - astart, NB)]], abb[i], asem[i])

    for i in range(_norm_bufs):
        @pl.when(i < nblk)
        def _():
            issue_loads(i, i)

    def pair_body(t, _):
        for i in range(_norm_bufs):
            k = 2 * t + i

            @pl.when(k < nblk)
            def _():
                ps = pstart(k)
                pltpu.make_async_copy(
                    x_hbm.at[pl.ds(ps, NB)], xb[i], xsem[i]).wait()
                pltpu.make_async_copy(
                    ab_hbm.at[idxall.at[pl.ds(ps - astart, NB)]],
                    abb[i], asem[i]).wait()

                @pl.when(k >= 2)
                def _():
                    pltpu.make_async_copy(
                        ob[i], out_hbm.at[pl.ds(pstart(k - 2), NB)],
                        osem[i]).wait()

                def row_body(r, _):
                    for f in range(FV):
                        xv = xb[i][r, pl.ds(f * L, L)]
                        av = abb[i][r, pl.ds(f * L, L)]
                        bv = abb[i][r, pl.ds(D + f * L, L)]
                        ob[i][r, pl.ds(f * L, L)] = xv * av + bv
                    return 0
                lax.fori_loop(0, NB, row_body, 0)

                pltpu.async_copy(ob[i], out_hbm.at[pl.ds(ps, NB)], osem[i])

                @pl.when(k + 2 < nblk)
                def _():
                    issue_loads(k + 2, i)
        return 0

    lax.fori_loop(0, NKB // 2, pair_body, 0)

    for i in range(_norm_bufs):
        @pl.when(i < nblk)
        def _():
            pltpu.make_async_copy(
                ob[i], out_hbm.at[pl.ds(pstart(0), NB)], osem[i]).wait()


@functools.cache
def _make_norm():
  mesh = plsc.VectorSubcoreMesh(
      core_axis_name="c", subcore_axis_name="s",
      num_cores=NC, num_subcores=NS)
  return functools.partial(
    pl.kernel,
    mesh=mesh,
    out_type=jax.ShapeDtypeStruct((N, D), _f32),
    scratch_types=[
        pltpu.VMEM((RPT,), _i32),
        pltpu.VMEM((NB, D), _f32),
        pltpu.VMEM((NB, D), _f32),
        pltpu.VMEM((NB, 2 * D), _f32),
        pltpu.VMEM((NB, 2 * D), _f32),
        pltpu.VMEM((NB, D), _f32),
        pltpu.VMEM((NB, D), _f32),
        pltpu.SemaphoreType.DMA,
        pltpu.SemaphoreType.DMA,
        pltpu.SemaphoreType.DMA,
        pltpu.SemaphoreType.DMA,
        pltpu.SemaphoreType.DMA,
        pltpu.SemaphoreType.DMA,
    ],
  )(_norm_body)


def kernel(x, batch, scale, bias):
    psum, psq, pcnt = _make_stats()(x, batch)
    ab = _finalize(psum, psq, pcnt, scale, bias)
    return _make_norm()(x, batch, ab)


# speedup vs baseline: 3.3427x; 1.2506x over previous
import functools

import jax
import jax.numpy as jnp
from jax import lax
from jax.experimental import pallas as pl
from jax.experimental.pallas import tpu as pltpu
from jax.experimental.pallas import tpu_sc as plsc

N = 50000
D = 256
S = 256
L = 16
NC = 2
NS = 16
NW = NC * NS
BR = 112
NBLK = (N + BR - 1) // BR
KMAX = (NBLK + NW - 1) // NW
H = D // 2
FH = H // L
FV = D // L

_f32 = jnp.float32
_i32 = jnp.int32


RPT = 1568
SBR = 32
SKB = RPT // SBR


def _stats_body(x_hbm, batch_hbm, psum, psq, pcnt,
                idxall, xb0, xb1, asum, asq, acnt, xs0, xs1):
    cid = lax.axis_index("c")
    sid = lax.axis_index("s")
    wid = sid * NC + cid
    xb = (xb0, xb1)
    xsem = (xs0, xs1)
    ones = jnp.ones((L,), _f32)
    sixteens = jnp.full((L,), 16.0, _f32)
    iota = lax.iota(_i32, L)

    base = RPT * wid
    tend = jnp.minimum(base + RPT, N)
    astart = pl.multiple_of(jnp.minimum(base, N - RPT), 8)
    pltpu.sync_copy(batch_hbm.at[pl.ds(astart, RPT)], idxall)
    nblk = (tend - base + SBR - 1) // SBR

    def pstart(k):
        return pl.multiple_of(jnp.minimum(base + SBR * k, N - SBR), 8)

    for h in range(2):
        def zero_body(r, _):
            for f in range(FH):
                asum[r, pl.ds(f * L, L)] = jnp.zeros((L,), _f32)
                asq[r, pl.ds(f * L, L)] = jnp.zeros((L,), _f32)
            if h == 0:
                acnt[r, pl.ds(0, L)] = jnp.zeros((L,), _f32)
            return 0
        lax.fori_loop(0, S + 1, zero_body, 0)

        def issue(k, i):
            pltpu.async_copy(
                x_hbm.at[pl.ds(pstart(k), SBR), pl.ds(h * H, H)],
                xb[i], xsem[i])

        for i in range(2):
            @pl.when(i < nblk)
            def _():
                issue(i, i)

        def pair_body(t, _):
            for i in range(2):
                k = 2 * t + i

                @pl.when(k < nblk)
                def _():
                    ps = pstart(k)
                    pltpu.make_async_copy(
                        x_hbm.at[pl.ds(ps, SBR), pl.ds(h * H, H)],
                        xb[i], xsem[i]).wait()
                    vfrom = base + SBR * k
                    vto = jnp.minimum(vfrom + SBR, tend)
                    loffk = ps - astart

                    def group_body(g, _):
                        bv = idxall[pl.ds(loffk + g * L, L)]
                        pos = ps + g * L + iota
                        validv = (pos >= vfrom) & (pos < vto)
                        u = jnp.where(validv, bv, S)
                        u0 = u[0]
                        uniform = (u0 == u[L - 1]) & (u0 < S)

                        @pl.when(uniform)
                        def _():
                            seg = u0
                            for f in range(FH):
                                acs = jnp.zeros((L,), _f32)
                                acq = jnp.zeros((L,), _f32)
                                for j in range(L):
                                    xv = xb[i][g * L + j, pl.ds(f * L, L)]
                                    acs = acs + xv
                                    acq = acq + xv * xv
                                plsc.addupdate(
                                    asum.at[seg, pl.ds(f * L, L)], acs)
                                plsc.addupdate(
                                    asq.at[seg, pl.ds(f * L, L)], acq)
                            if h == 0:
                                plsc.addupdate(
                                    acnt.at[seg, pl.ds(0, L)], sixteens)

                        @pl.when(jnp.logical_not(uniform))
                        def _():
                            for j in range(L):
                                seg = u[j]
                                for f in range(FH):
                                    xv = xb[i][g * L + j, pl.ds(f * L, L)]
                                    plsc.addupdate(
                                        asum.at[seg, pl.ds(f * L, L)], xv)
                                    plsc.addupdate(
                                        asq.at[seg, pl.ds(f * L, L)],
                                        xv * xv)
                                if h == 0:
                                    plsc.addupdate(
                                        acnt.at[seg, pl.ds(0, L)], ones)
                        return 0
                    lax.fori_loop(0, SBR // L, group_body, 0)

                    @pl.when(k + 2 < nblk)
                    def _():
                        issue(k + 2, i)
            return 0

        lax.fori_loop(0, (SKB + 1) // 2, pair_body, 0)

        pltpu.sync_copy(asum, psum.at[h, wid])
        pltpu.sync_copy(asq, psq.at[h, wid])
        if h == 0:
            pltpu.sync_copy(acnt, pcnt.at[wid])


@functools.cache
def _make_stats():
  mesh = plsc.VectorSubcoreMesh(
      core_axis_name="c", subcore_axis_name="s",
      num_cores=NC, num_subcores=NS)
  return functools.partial(
    pl.kernel,
    mesh=mesh,
    out_type=(
        jax.ShapeDtypeStruct((2, NW, S + 1, H), _f32),
        jax.ShapeDtypeStruct((2, NW, S + 1, H), _f32),
        jax.ShapeDtypeStruct((NW, S + 1, L), _f32),
    ),
    scratch_types=[
        pltpu.VMEM((RPT,), _i32),
        pltpu.VMEM((SBR, H), _f32),
        pltpu.VMEM((SBR, H), _f32),
        pltpu.VMEM((S + 1, H), _f32),
        pltpu.VMEM((S + 1, H), _f32),
        pltpu.VMEM((S + 1, L), _f32),
        pltpu.SemaphoreType.DMA,
        pltpu.SemaphoreType.DMA,
    ],
  )(_stats_body)


def _finalize_body(psum_ref, psq_ref, pcnt_ref, scale_ref, bias_ref, ab_ref):
    s0 = jnp.sum(psum_ref[0], axis=0)[:S]
    s1 = jnp.sum(psum_ref[1], axis=0)[:S]
    q0 = jnp.sum(psq_ref[0], axis=0)[:S]
    q1 = jnp.sum(psq_ref[1], axis=0)[:S]
    s = jnp.concatenate([s0, s1], axis=1)
    q = jnp.concatenate([q0, q1], axis=1)
    c = jnp.sum(pcnt_ref[...], axis=0)[:S, 0:1]
    c_safe = jnp.maximum(c, 1.0)
    mean = s / c_safe
    denom = jnp.maximum(c - 1.0, 1.0)
    var = jnp.maximum((q - c * mean * mean) / denom, 0.0)
    std = jnp.sqrt(var)
    a = scale_ref[...][None, :] / (std + 1e-5)
    bb = bias_ref[...][None, :] - mean * a
    ab_ref[:, 0:D] = a
    ab_ref[:, D:2 * D] = bb


def _finalize(psum, psq, pcnt, scale, bias):
    return pl.pallas_call(
        _finalize_body,
        out_shape=jax.ShapeDtypeStruct((S, 2 * D), _f32),
    )(psum, psq, pcnt, scale, bias)


RPT = 1568
NB = 56
NKB = RPT // NB
_norm_bufs = 2


def _norm_body(x_hbm, batch_hbm, ab_hbm, out_hbm,
               idxall, xb0, xb1, ab0, ab1, ob0, ob1,
               xs0, xs1, as0, as1, os0, os1):
    cid = lax.axis_index("c")
    sid = lax.axis_index("s")
    wid = sid * NC + cid
    xb = (xb0, xb1)
    abb = (ab0, ab1)
    ob = (ob0, ob1)
    xsem = (xs0, xs1)
    asem = (as0, as1)
    osem = (os0, os1)

    base = RPT * wid
    astart = pl.multiple_of(jnp.minimum(base, N - RPT), 8)
    pltpu.sync_copy(batch_hbm.at[pl.ds(astart, RPT)], idxall)
    nblk = (jnp.minimum(N - base, RPT) + NB - 1) // NB

    def pstart(k):
        return pl.multiple_of(jnp.minimum(base + NB * k, N - NB), 8)

    def issue_loads(k, i):
        ps = pstart(k)
        pltpu.async_copy(x_hbm.at[pl.ds(ps, NB)], xb[i], xsem[i])
        pltpu.async_copy(
            ab_hbm.at[idxall.at[pl.ds(ps - astart, NB)]], abb[i], asem[i])

    for i in range(_norm_bufs):
        @pl.when(i < nblk)
        def _():
            issue_loads(i, i)

    def pair_body(t, _):
        for i in range(_norm_bufs):
            k = 2 * t + i

            @pl.when(k < nblk)
            def _():
                ps = pstart(k)
                pltpu.make_async_copy(
                    x_hbm.at[pl.ds(ps, NB)], xb[i], xsem[i]).wait()
                pltpu.make_async_copy(
                    ab_hbm.at[idxall.at[pl.ds(ps - astart, NB)]],
                    abb[i], asem[i]).wait()

                @pl.when(k >= 2)
                def _():
                    pltpu.make_async_copy(
                        ob[i], out_hbm.at[pl.ds(pstart(k - 2), NB)],
                        osem[i]).wait()

                def row_body(r, _):
                    for f in range(FV):
                        xv = xb[i][r, pl.ds(f * L, L)]
                        av = abb[i][r, pl.ds(f * L, L)]
                        bv = abb[i][r, pl.ds(D + f * L, L)]
                        ob[i][r, pl.ds(f * L, L)] = xv * av + bv
                    return 0
                lax.fori_loop(0, NB, row_body, 0)

                pltpu.async_copy(ob[i], out_hbm.at[pl.ds(ps, NB)], osem[i])

                @pl.when(k + 2 < nblk)
                def _():
                    issue_loads(k + 2, i)
        return 0

    lax.fori_loop(0, NKB // 2, pair_body, 0)

    for i in range(_norm_bufs):
        @pl.when(i < nblk)
        def _():
            pltpu.make_async_copy(
                ob[i], out_hbm.at[pl.ds(pstart(0), NB)], osem[i]).wait()


@functools.cache
def _make_norm():
  mesh = plsc.VectorSubcoreMesh(
      core_axis_name="c", subcore_axis_name="s",
      num_cores=NC, num_subcores=NS)
  return functools.partial(
    pl.kernel,
    mesh=mesh,
    out_type=jax.ShapeDtypeStruct((N, D), _f32),
    scratch_types=[
        pltpu.VMEM((RPT,), _i32),
        pltpu.VMEM((NB, D), _f32),
        pltpu.VMEM((NB, D), _f32),
        pltpu.VMEM((NB, 2 * D), _f32),
        pltpu.VMEM((NB, 2 * D), _f32),
        pltpu.VMEM((NB, D), _f32),
        pltpu.VMEM((NB, D), _f32),
        pltpu.SemaphoreType.DMA,
        pltpu.SemaphoreType.DMA,
        pltpu.SemaphoreType.DMA,
        pltpu.SemaphoreType.DMA,
        pltpu.SemaphoreType.DMA,
        pltpu.SemaphoreType.DMA,
    ],
  )(_norm_body)


def kernel(x, batch, scale, bias):
    psum, psq, pcnt = _make_stats()(x, batch)
    ab = _finalize(psum, psq, pcnt, scale, bias)
    return _make_norm()(x, batch, ab)
